# baseline (speedup 1.0000x reference)
import jax, jax.numpy as jnp
import numpy as np
from jax.experimental import pallas as pl

N = 8192
E = 131072
H = 64
TOP_K = 32
NHEADS = 4


def _gcn_conv(x, ei, w, b):
    n = x.shape[0]
    sl = jnp.arange(n, dtype=ei.dtype)
    row = jnp.concatenate([ei[0], sl])
    col = jnp.concatenate([ei[1], sl])
    deg = jnp.zeros((n,), x.dtype).at[col].add(1.0)
    dis = 1.0 / jnp.sqrt(jnp.maximum(deg, 1.0))
    norm = dis[row] * dis[col]
    h = x @ w
    out = jnp.zeros((n, w.shape[1]), x.dtype).at[col].add(norm[:, None] * h[row])
    return out + b


def _mha(h, p):
    n = h.shape[0]
    dh = H // NHEADS
    q = (h @ p['mha_wq'] + p['mha_bq']).reshape(n, NHEADS, dh).transpose(1, 0, 2)
    k = (h @ p['mha_wk'] + p['mha_bk']).reshape(n, NHEADS, dh).transpose(1, 0, 2)
    v = (h @ p['mha_wv'] + p['mha_bv']).reshape(n, NHEADS, dh).transpose(1, 0, 2)
    scores = jnp.einsum('hnd,hmd->hnm', q, k) / np.sqrt(dh)
    attn = jax.nn.softmax(scores, axis=-1)
    o = jnp.einsum('hnm,hmd->hnd', attn, v).transpose(1, 0, 2).reshape(n, H)
    return o @ p['mha_wo'] + p['mha_bo']


def _cdist(c):
    sq = jnp.sum(c * c, axis=1)
    d2 = jnp.maximum(sq[:, None] + sq[None, :] - 2.0 * (c @ c.T), 0.0)
    d = jnp.sqrt(jnp.where(d2 <= 0.0, 1.0, d2))
    return jnp.where(d2 <= 0.0, 0.0, d)


def _mlp_kernel(cf_ref, w1_ref, b1_ref, w2_ref, b2_ref, w3_ref, b3_ref, o_ref):
    h = jax.nn.relu(cf_ref[...] @ w1_ref[...] + b1_ref[...])
    h = jax.nn.relu(h @ w2_ref[...] + b2_ref[...])
    o_ref[...] = h @ w3_ref[...] + b3_ref[...]


def kernel(x, params, edge_index):
    p = params
    coords = x[:, -2:]
    feat = x[:, :-2]
    hl = jnp.concatenate([feat @ p['lp_w'] + p['lp_b'], coords @ p['lc_w'] + p['lc_b']], axis=-1)
    hl = jax.nn.relu(_gcn_conv(hl, edge_index, p['lg_w'], p['lg_b']))
    f_local = hl @ p['lf_w'] + p['lf_b']
    hm = jnp.concatenate([feat @ p['mp_w'] + p['mp_b'], coords @ p['mc_w'] + p['mc_b']], axis=-1)
    hm = jax.nn.relu(_gcn_conv(hm, edge_index, p['mg1_w'], p['mg1_b']))
    hm = jax.nn.relu(_gcn_conv(hm, edge_index, p['mg2_w'], p['mg2_b']))
    hm = jax.nn.relu(_gcn_conv(hm, edge_index, p['mg3_w'], p['mg3_b']))
    hm = _mha(hm, p)
    f_medium = hm @ p['mf_w'] + p['mf_b']
    hn = feat @ p['gp_w'] + p['gp_b']
    hc = coords @ p['gc_w'] + p['gc_b']
    hcat = jnp.concatenate([hn, hc], axis=-1)
    Q = hcat @ p['gq_w'] + p['gq_b']
    K = hcat @ p['gk_w'] + p['gk_b']
    V = hcat @ p['gv_w'] + p['gv_b']
    dist = _cdist(coords)
    negd, idx = jax.lax.top_k(-dist, TOP_K)
    d_topk = -negd
    k_topk = K[idx]
    v_topk = V[idx]
    scores = jnp.einsum('nd,nkd->nk', Q, k_topk) / np.sqrt(H) + 1.0 / (d_topk + 1e-06)
    attn = jax.nn.softmax(scores, axis=-1)
    gi = jnp.einsum('nk,nkd->nd', attn, v_topk)
    f_global = gi @ p['gf_w'] + p['gf_b']
    w = jax.nn.softmax(p['fw'])
    cf = jnp.concatenate([w[0] * f_local, w[1] * f_medium, w[2] * f_global], axis=-1)
    out = pl.pallas_call(
        _mlp_kernel,
        out_shape=jax.ShapeDtypeStruct((N, 2), jnp.float32),
    )(cf, p['fc1_w'], p['fc1_b'], p['fc2_w'], p['fc2_b'], p['fc3_w'], p['fc3_b'])
    return out


# speedup vs baseline: 6.1552x; 6.1552x over previous
import functools
import numpy as np
import jax
import jax.numpy as jnp
from jax import lax
from jax.experimental import pallas as pl
from jax.experimental.pallas import tpu as pltpu
from jax.experimental.pallas import tpu_sc as plsc

NN = 8192
EE = 131072
H = 64
TOP_K = 32
NHEADS = 4
DH = H // NHEADS

NC = 2
NS = 16
NW = NC * NS
CH = 128

BROW = 512
BTOP = 256


def _encoder_body(x_ref, w0_ref, b0_ref, wq_ref, bq_ref, wkv_ref, bkv_ref,
                  h0_ref, q_ref, kv_ref):
    xb = x_ref[...]
    h0_ref[...] = xb @ w0_ref[...] + b0_ref[...]
    q_ref[...] = xb @ wq_ref[...] + bq_ref[...]
    kv_ref[...] = xb @ wkv_ref[...] + bkv_ref[...]


def _pre2_body(h0_ref, deg_ref, wl_ref, wm_ref, hh_ref, g_ref):
    deg = deg_ref[...] + 1.0
    dis = 1.0 / jnp.sqrt(jnp.maximum(deg, 1.0))
    h0 = h0_ref[...]
    hl = h0[:, :80] @ wl_ref[...]
    hm = h0[:, 80:] @ wm_ref[...]
    hh_ref[:, :H] = hl
    hh_ref[:, H:] = hm
    g_ref[:, :H] = dis * hl
    g_ref[:, H:] = dis * hm


def _post_step_body(acc_ref, hh_ref, deg_ref, bl_ref, bm_ref, w2_ref,
                    hl_ref, hh2_ref, g2_ref):
    deg = deg_ref[...] + 1.0
    dis = 1.0 / jnp.sqrt(jnp.maximum(deg, 1.0))
    sn = dis * dis
    acc = acc_ref[0] + acc_ref[1]
    hh = hh_ref[...]
    hl = jnp.maximum(dis * acc[:, :H] + sn * hh[:, :H] + bl_ref[...], 0.0)
    hm1 = jnp.maximum(dis * acc[:, H:] + sn * hh[:, H:] + bm_ref[...], 0.0)
    hl_ref[...] = hl
    h2 = hm1 @ w2_ref[...]
    hh2_ref[...] = h2
    g2_ref[:, :H] = dis * h2
    g2_ref[:, H:] = jnp.zeros((h2.shape[0], H), jnp.float32)


def _step_body(acc_ref, hh_ref, deg_ref, b_ref, w_ref, hh3_ref, g3_ref):
    deg = deg_ref[...] + 1.0
    dis = 1.0 / jnp.sqrt(jnp.maximum(deg, 1.0))
    sn = dis * dis
    acc = acc_ref[0, :, :H] + acc_ref[1, :, :H]
    hm = jnp.maximum(dis * acc + sn * hh_ref[...] + b_ref[...], 0.0)
    h3 = hm @ w_ref[...]
    hh3_ref[...] = h3
    g3_ref[:, :H] = dis * h3
    g3_ref[:, H:] = jnp.zeros((h3.shape[0], H), jnp.float32)


def _post_qkv_body(acc_ref, hh_ref, deg_ref, b_ref, wqkv_ref, bqkv_ref,
                   qkv_ref):
    deg = deg_ref[...] + 1.0
    dis = 1.0 / jnp.sqrt(jnp.maximum(deg, 1.0))
    sn = dis * dis
    acc = acc_ref[0, :, :H] + acc_ref[1, :, :H]
    hm = jnp.maximum(dis * acc + sn * hh_ref[...] + b_ref[...], 0.0)
    qkv_ref[...] = hm @ wqkv_ref[...] + bqkv_ref[...]


def _mha_body(qkv_ref, qkvf_ref, wo_ref, bo_ref, fmed_ref):
    qb = qkv_ref[...]
    outs = []
    for h in range(NHEADS):
        qh = qb[:, h * DH:(h + 1) * DH]
        kh = qkvf_ref[:, H + h * DH:H + (h + 1) * DH]
        vh = qkvf_ref[:, 2 * H + h * DH:2 * H + (h + 1) * DH]
        s = lax.dot_general(qh, kh, (((1,), (1,)), ((), ())),
                            preferred_element_type=jnp.float32) * 0.25
        m = jnp.max(s, axis=1, keepdims=True)
        e = jnp.exp(s - m)
        a = e / jnp.sum(e, axis=1, keepdims=True)
        outs.append(jnp.dot(a, vh, preferred_element_type=jnp.float32))
    o = jnp.concatenate(outs, axis=1)
    fmed_ref[...] = o @ wo_ref[...] + bo_ref[...]


def _topk_body(cb_ref, ct_ref, dk_ref, idx_ref):
    cb = cb_ref[...]
    ct = ct_ref[...]
    sqb = jnp.sum(cb * cb, axis=1, keepdims=True)
    sqa = jnp.sum(ct * ct, axis=0, keepdims=True)
    prod = lax.dot_general(cb, ct, (((1,), (0,)), ((), ())),
                           preferred_element_type=jnp.float32)
    d2 = jnp.maximum(sqb + sqa - 2.0 * prod, 0.0)
    iota = lax.broadcasted_iota(jnp.int32, (1, NN), 1)
    inf = jnp.float32(np.inf)
    for k in range(TOP_K):
        m = jnp.min(d2, axis=1, keepdims=True)
        sel = jnp.min(jnp.where(d2 == m, iota, NN), axis=1, keepdims=True)
        dk_ref[:, k:k + 1] = jnp.where(
            m <= 0.0, 0.0, jnp.sqrt(jnp.where(m <= 0.0, 1.0, m)))
        idx_ref[:, k:k + 1] = sel
        d2 = jnp.where(iota == sel, inf, d2)


def _combine_body(q_ref, ktvt_ref, dk_ref, hl_ref, fmed_ref,
                  lfw_ref, lfb_ref, gfw_ref, gfb_ref,
                  w1_ref, b1_ref, w2_ref, b2_ref, w3_ref, b3_ref, o_ref):
    qv = q_ref[...]
    ktvt = ktvt_ref[...]
    kt = ktvt[:, :, :H]
    vt = ktvt[:, :, H:]
    dk = dk_ref[...]
    s = jnp.sum(qv[:, None, :] * kt, axis=2) * 0.125
    s = s + 1.0 / (dk + 1e-6)
    m = jnp.max(s, axis=1, keepdims=True)
    e = jnp.exp(s - m)
    attn = e / jnp.sum(e, axis=1, keepdims=True)
    gi = jnp.sum(attn[:, :, None] * vt, axis=1)
    f_g = gi @ gfw_ref[...] + gfb_ref[...]
    f_l = hl_ref[...] @ lfw_ref[...] + lfb_ref[...]
    cf = jnp.concatenate([f_l, fmed_ref[...], f_g], axis=1)
    h2 = jnp.maximum(cf @ w1_ref[...] + b1_ref[...], 0.0)
    h2 = jnp.maximum(h2 @ w2_ref[...] + b2_ref[...], 0.0)
    o_ref[...] = h2 @ w3_ref[...] + b3_ref[...]


def _full(shape):
    nd = len(shape)
    return pl.BlockSpec(shape, lambda i, *_: (0,) * nd)


def _rows(shape):
    nd = len(shape)
    return pl.BlockSpec(shape, lambda i: (i,) + (0,) * (nd - 1))


def _mesh():
    return plsc.VectorSubcoreMesh(core_axis_name="c", subcore_axis_name="s")


def _deg_kernel(col2):
    per_w = EE // NW
    n_ch = per_w // CH
    rows_pt = NN // NS

    @functools.partial(
        pl.kernel,
        out_type=jax.ShapeDtypeStruct((NC, NN, 128), jnp.float32),
        mesh=_mesh(),
        scratch_types=[
            pltpu.VMEM((n_ch, CH), jnp.int32),
            pltpu.VMEM((CH, 128), jnp.float32),
            pltpu.VMEM_SHARED((NN, 128), jnp.float32),
        ],
    )
    def k(col_hbm, out_hbm, cidx_v, ones_v, acc):
        cid = lax.axis_index("c")
        sid = lax.axis_index("s")
        wid = sid * NC + cid

        def fill(i, val):
            for j in range(128 // 16):
                ones_v[i, pl.ds(j * 16, 16)] = jnp.full((16,), val, jnp.float32)
            return val

        lax.fori_loop(0, CH, fill, jnp.float32(0.0))
        for t in range(rows_pt // CH):
            pltpu.sync_copy(ones_v, acc.at[pl.ds(sid * rows_pt + t * CH, CH)])
        plsc.subcore_barrier()
        lax.fori_loop(0, CH, fill, jnp.float32(1.0))
        pltpu.sync_copy(col_hbm.at[pl.ds(wid * n_ch, n_ch)], cidx_v)
        for j in range(n_ch):
            pltpu.sync_copy(ones_v, acc.at[cidx_v.at[j]], add=True)
        plsc.subcore_barrier()
        pltpu.sync_copy(acc.at[pl.ds(sid * rows_pt, rows_pt)],
                        out_hbm.at[cid].at[pl.ds(sid * rows_pt, rows_pt)])

    return k(col2)


def _edge_agg(g, row2, col2):
    per_w = EE // NW
    n_ch = per_w // CH
    rows_pt = NN // NS

    @functools.partial(
        pl.kernel,
        out_type=jax.ShapeDtypeStruct((NC, NN, 128), jnp.float32),
        mesh=_mesh(),
        scratch_types=[
            pltpu.VMEM((n_ch, CH), jnp.int32),
            pltpu.VMEM((n_ch, CH), jnp.int32),
            pltpu.VMEM((CH, 128), jnp.float32),
            pltpu.VMEM_SHARED((NN, 128), jnp.float32),
            pltpu.SemaphoreType.DMA,
        ],
    )
    def k(g_hbm, row_hbm, col_hbm, out_hbm, ridx_v, cidx_v, rows_v, acc, sem):
        cid = lax.axis_index("c")
        sid = lax.axis_index("s")
        wid = sid * NC + cid

        def zero(i, carry):
            for j in range(128 // 16):
                rows_v[i, pl.ds(j * 16, 16)] = jnp.zeros((16,), jnp.float32)
            return carry
        lax.fori_loop(0, CH, zero, 0)
        for t in range(rows_pt // CH):
            pltpu.sync_copy(rows_v, acc.at[pl.ds(sid * rows_pt + t * CH, CH)])
        plsc.subcore_barrier()

        pltpu.sync_copy(row_hbm.at[pl.ds(wid * n_ch, n_ch)], ridx_v)
        pltpu.sync_copy(col_hbm.at[pl.ds(wid * n_ch, n_ch)], cidx_v)
        for j in range(n_ch):
            pltpu.async_copy(g_hbm.at[ridx_v.at[j]], rows_v, sem).wait()
            pltpu.sync_copy(rows_v, acc.at[cidx_v.at[j]], add=True)
        plsc.subcore_barrier()
        pltpu.sync_copy(acc.at[pl.ds(sid * rows_pt, rows_pt)],
                        out_hbm.at[cid].at[pl.ds(sid * rows_pt, rows_pt)])

    return k(g, row2, col2)


def _gather_kv(kv, idx2):
    total = NN * TOP_K
    per_w = total // NW
    n_ch = per_w // CH
    sup = 512
    n_sup = per_w // sup

    @functools.partial(
        pl.kernel,
        out_type=jax.ShapeDtypeStruct((total, 2 * H), jnp.float32),
        mesh=_mesh(),
        scratch_types=[
            pltpu.VMEM((n_ch, CH), jnp.int32),
            pltpu.VMEM((sup, 2 * H), jnp.float32),
            pltpu.SemaphoreType.DMA,
        ],
    )
    def k(kv_hbm, idx_hbm, out_hbm, idx_v, rows_v, sem):
        cid = lax.axis_index("c")
        sid = lax.axis_index("s")
        wid = sid * NC + cid
        pltpu.sync_copy(idx_hbm.at[pl.ds(wid * n_ch, n_ch)], idx_v)
        for s in range(n_sup):
            descs = []
            for j in range(sup // CH):
                row = s * (sup // CH) + j
                descs.append(pltpu.async_copy(
                    kv_hbm.at[idx_v.at[row]],
                    rows_v.at[pl.ds(j * CH, CH)], sem))
            for dsc in descs:
                dsc.wait()
            pltpu.sync_copy(
                rows_v, out_hbm.at[pl.ds(wid * per_w + s * sup, sup)])

    return k(kv, idx2)


def kernel(x, params, edge_index):
    p = params
    f32 = jnp.float32

    w0 = jnp.zeros((128, 160), f32)
    w0 = w0.at[:126, :64].set(p['lp_w']).at[126:, 64:80].set(p['lc_w'])
    w0 = w0.at[:126, 80:144].set(p['mp_w']).at[126:, 144:160].set(p['mc_w'])
    b0 = jnp.concatenate([p['lp_b'], p['lc_b'], p['mp_b'], p['mc_b']])[None]
    pg = jnp.zeros((128, 80), f32)
    pg = pg.at[:126, :64].set(p['gp_w']).at[126:, 64:].set(p['gc_w'])
    bg = jnp.concatenate([p['gp_b'], p['gc_b']])
    wq = pg @ p['gq_w']
    bq = (bg @ p['gq_w'] + p['gq_b'])[None]
    wkv = jnp.concatenate([pg @ p['gk_w'], pg @ p['gv_w']], axis=1)
    bkv = jnp.concatenate([bg @ p['gk_w'] + p['gk_b'],
                           bg @ p['gv_w'] + p['gv_b']])[None]
    wqkv = jnp.concatenate([p['mha_wq'], p['mha_wk'], p['mha_wv']], axis=1)
    bqkv = jnp.concatenate([p['mha_bq'], p['mha_bk'], p['mha_bv']])[None]
    wo2 = p['mha_wo'] @ p['mf_w']
    bo2 = (p['mha_bo'] @ p['mf_w'] + p['mf_b'])[None]
    wmix = jax.nn.softmax(p['fw'])
    w1s = p['fc1_w'] * jnp.repeat(wmix, 2)[:, None]

    row2 = edge_index[0].reshape(EE // CH, CH)
    col2 = edge_index[1].reshape(EE // CH, CH)

    nblk = NN // BROW
    h0, q, kv = pl.pallas_call(
        _encoder_body,
        grid=(nblk,),
        in_specs=[_rows((BROW, 128)), _full((128, 160)), _full((1, 160)),
                  _full((128, 64)), _full((1, 64)),
                  _full((128, 128)), _full((1, 128))],
        out_specs=[_rows((BROW, 160)), _rows((BROW, 64)), _rows((BROW, 128))],
        out_shape=[jax.ShapeDtypeStruct((NN, 160), f32),
                   jax.ShapeDtypeStruct((NN, 64), f32),
                   jax.ShapeDtypeStruct((NN, 128), f32)],
    )(x, w0, b0, wq, bq, wkv, bkv)

    dego = _deg_kernel(col2)
    deg_e = dego[0, :, :1] + dego[1, :, :1]

    hh, g = pl.pallas_call(
        _pre2_body,
        grid=(nblk,),
        in_specs=[_rows((BROW, 160)), _rows((BROW, 1)),
                  _full((80, 64)), _full((80, 64))],
        out_specs=[_rows((BROW, 128)), _rows((BROW, 128))],
        out_shape=[jax.ShapeDtypeStruct((NN, 128), f32),
                   jax.ShapeDtypeStruct((NN, 128), f32)],
    )(h0, deg_e, p['lg_w'], p['mg1_w'])

    acc1 = _edge_agg(g, row2, col2)

    hl, hh2, g2 = pl.pallas_call(
        _post_step_body,
        grid=(nblk,),
        in_specs=[pl.BlockSpec((NC, BROW, 128), lambda i: (0, i, 0)),
                  _rows((BROW, 128)), _rows((BROW, 1)),
                  _full((1, 64)), _full((1, 64)), _full((64, 64))],
        out_specs=[_rows((BROW, 64)), _rows((BROW, 64)), _rows((BROW, 128))],
        out_shape=[jax.ShapeDtypeStruct((NN, 64), f32),
                   jax.ShapeDtypeStruct((NN, 64), f32),
                   jax.ShapeDtypeStruct((NN, 128), f32)],
    )(acc1, hh, deg_e, p['lg_b'][None], p['mg1_b'][None], p['mg2_w'])

    acc2 = _edge_agg(g2, row2, col2)

    hh3, g3 = pl.pallas_call(
        _step_body,
        grid=(nblk,),
        in_specs=[pl.BlockSpec((NC, BROW, 128), lambda i: (0, i, 0)),
                  _rows((BROW, 64)), _rows((BROW, 1)),
                  _full((1, 64)), _full((64, 64))],
        out_specs=[_rows((BROW, 64)), _rows((BROW, 128))],
        out_shape=[jax.ShapeDtypeStruct((NN, 64), f32),
                   jax.ShapeDtypeStruct((NN, 128), f32)],
    )(acc2, hh2, deg_e, p['mg2_b'][None], p['mg3_w'])

    acc3 = _edge_agg(g3, row2, col2)

    qkv = pl.pallas_call(
        _post_qkv_body,
        grid=(nblk,),
        in_specs=[pl.BlockSpec((NC, BROW, 128), lambda i: (0, i, 0)),
                  _rows((BROW, 64)), _rows((BROW, 1)),
                  _full((1, 64)), _full((64, 192)), _full((1, 192))],
        out_specs=_rows((BROW, 192)),
        out_shape=jax.ShapeDtypeStruct((NN, 192), f32),
    )(acc3, hh3, deg_e, p['mg3_b'][None], wqkv, bqkv)

    f_med = pl.pallas_call(
        _mha_body,
        grid=(nblk,),
        in_specs=[_rows((BROW, 192)), _full((NN, 192)),
                  _full((64, 2)), _full((1, 2))],
        out_specs=_rows((BROW, 2)),
        out_shape=jax.ShapeDtypeStruct((NN, 2), f32),
    )(qkv, qkv, wo2, bo2)

    coords = x[:, 126:128]
    ct = coords.T
    dk, idx = pl.pallas_call(
        _topk_body,
        grid=(NN // BTOP,),
        in_specs=[_rows((BTOP, 2)), _full((2, NN))],
        out_specs=[_rows((BTOP, TOP_K)), _rows((BTOP, TOP_K))],
        out_shape=[jax.ShapeDtypeStruct((NN, TOP_K), f32),
                   jax.ShapeDtypeStruct((NN, TOP_K), jnp.int32)],
    )(coords, ct)

    idx2 = idx.reshape(NN * TOP_K // CH, CH)
    ktvt = _gather_kv(kv, idx2).reshape(NN, TOP_K, 2 * H)

    out = pl.pallas_call(
        _combine_body,
        grid=(nblk,),
        in_specs=[_rows((BROW, 64)),
                  pl.BlockSpec((BROW, TOP_K, 2 * H), lambda i: (i, 0, 0)),
                  _rows((BROW, TOP_K)), _rows((BROW, 64)), _rows((BROW, 2)),
                  _full((64, 2)), _full((1, 2)), _full((64, 2)), _full((1, 2)),
                  _full((6, 64)), _full((1, 64)), _full((64, 32)),
                  _full((1, 32)), _full((32, 2)), _full((1, 2))],
        out_specs=_rows((BROW, 2)),
        out_shape=jax.ShapeDtypeStruct((NN, 2), f32),
    )(q, ktvt, dk, hl, f_med,
      p['lf_w'], p['lf_b'][None], p['gf_w'], p['gf_b'][None],
      w1s, p['fc1_b'][None], p['fc2_w'], p['fc2_b'][None],
      p['fc3_w'], p['fc3_b'][None])

    return out
